# Initial kernel scaffold; baseline (speedup 1.0000x reference)
import jax
import jax.numpy as jnp
from jax.experimental import pallas as pl


def kernel(node_positions, adjacency, edge_index):
    raise NotImplementedError("write your pallas kernel here")


# speedup vs baseline: 3.4436x; 3.4436x over previous
import jax
import jax.numpy as jnp
from jax.experimental import pallas as pl
from jax.experimental.pallas import tpu as pltpu

EPS = 0.001
PROX = 0.15
RB = 256
CB = 256


def _body(posx_r_ref, posy_r_ref, posx_c_ref, posy_c_ref,
          src_c_ref, dst_c_ref, src_r_ref, dst_r_ref, out_ref):
    f32 = jnp.float32
    posx_r = posx_r_ref[...]
    posy_r = posy_r_ref[...]
    posx_c = posx_c_ref[...]
    posy_c = posy_c_ref[...]
    src_c = src_c_ref[...]
    dst_c = dst_c_ref[...]
    src_r = src_r_ref[...]
    dst_r = dst_r_ref[...]
    n = posx_r.shape[1]
    e = src_c.shape[0]

    gb = 512
    lanes = jax.lax.broadcasted_iota(jnp.int32, (gb, n), 1)
    subl = jax.lax.broadcasted_iota(jnp.int32, (n, gb), 0)

    def gather_col(idx_c):
        xs, ys = [], []
        for g0 in range(0, e, gb):
            m = (lanes == idx_c[g0:g0 + gb]).astype(f32)
            xs.append(jnp.sum(m * posx_r, axis=1, keepdims=True))
            ys.append(jnp.sum(m * posy_r, axis=1, keepdims=True))
        return jnp.concatenate(xs, axis=0), jnp.concatenate(ys, axis=0)

    def gather_row(idx_r):
        xs, ys = [], []
        for g0 in range(0, e, gb):
            m = (subl == idx_r[:, g0:g0 + gb]).astype(f32)
            xs.append(jnp.sum(m * posx_c, axis=0, keepdims=True))
            ys.append(jnp.sum(m * posy_c, axis=0, keepdims=True))
        return jnp.concatenate(xs, axis=1), jnp.concatenate(ys, axis=1)

    a1x_c, a1y_c = gather_col(src_c)
    a2x_c, a2y_c = gather_col(dst_c)
    b1x_r, b1y_r = gather_row(src_r)
    b2x_r, b2y_r = gather_row(dst_r)

    d1x_c = a2x_c - a1x_c
    d1y_c = a2y_c - a1y_c
    midx_c = (a1x_c + a2x_c) * 0.5
    midy_c = (a1y_c + a2y_c) * 0.5
    lsq_c = d1x_c * d1x_c + d1y_c * d1y_c
    aa_c = jnp.maximum(lsq_c, 1e-12)
    inv_a_c = 1.0 / aa_c
    hl_c = jnp.sqrt(jnp.maximum(lsq_c, 1e-24)) * 0.5

    d2x_r = b2x_r - b1x_r
    d2y_r = b2y_r - b1y_r
    midx_r = (b1x_r + b2x_r) * 0.5
    midy_r = (b1y_r + b2y_r) * 0.5
    lsq_r = d2x_r * d2x_r + d2y_r * d2y_r
    ee_r = jnp.maximum(lsq_r, 1e-12)
    inv_e_r = 1.0 / ee_r
    hl_r = jnp.sqrt(jnp.maximum(lsq_r, 1e-24)) * 0.5

    nb_r = e // RB
    nb_c = e // CB
    acc_loss = f32(0.0)
    acc_cnt = f32(0.0)
    for bi in range(nb_r):
        r0 = bi * RB
        A1x = a1x_c[r0:r0 + RB]
        A1y = a1y_c[r0:r0 + RB]
        D1x = d1x_c[r0:r0 + RB]
        D1y = d1y_c[r0:r0 + RB]
        MIx = midx_c[r0:r0 + RB]
        MIy = midy_c[r0:r0 + RB]
        AA = aa_c[r0:r0 + RB]
        IA = inv_a_c[r0:r0 + RB]
        HI = hl_c[r0:r0 + RB]
        SI = src_c[r0:r0 + RB]
        DI = dst_c[r0:r0 + RB]
        for bj in range(bi, nb_c):
            c0 = bj * CB
            B1x = b1x_r[:, c0:c0 + CB]
            B1y = b1y_r[:, c0:c0 + CB]
            D2x = d2x_r[:, c0:c0 + CB]
            D2y = d2y_r[:, c0:c0 + CB]
            MJx = midx_r[:, c0:c0 + CB]
            MJy = midy_r[:, c0:c0 + CB]
            EE = ee_r[:, c0:c0 + CB]
            IE = inv_e_r[:, c0:c0 + CB]
            HJ = hl_r[:, c0:c0 + CB]
            SJ = src_r[:, c0:c0 + CB]
            DJ = dst_r[:, c0:c0 + CB]

            rx = A1x - B1x
            ry = A1y - B1y
            b = D1x * D2x + D1y * D2y
            c = D1x * rx + D1y * ry
            f = D2x * rx + D2y * ry
            denom = jnp.maximum(AA * EE - b * b, 1e-12)
            rden = 1.0 / denom
            s = jnp.clip((b * f - c * EE) * rden, 0.0, 1.0)
            t = jnp.clip((b * s + f) * IE, 0.0, 1.0)
            s = jnp.clip((b * t - c) * IA, 0.0, 1.0)
            dx = rx + s * D1x - t * D2x
            dy = ry + s * D1y - t * D2y
            dist = jnp.sqrt(jnp.maximum(dx * dx + dy * dy, 1e-24))

            mdx = MIx - MJx
            mdy = MIy - MJy
            reach = HI + HJ + PROX
            prox = (mdx * mdx + mdy * mdy) < (reach * reach)
            share = ((SI == SJ) | (SI == DJ) | (DI == SJ) | (DI == DJ))
            cand = prox & jnp.logical_not(share)
            if bi == bj:
                ii = jax.lax.broadcasted_iota(jnp.int32, (RB, CB), 0)
                jj = jax.lax.broadcasted_iota(jnp.int32, (RB, CB), 1)
                cand = cand & (jj > ii)
            contrib = jnp.where(cand, jnp.maximum(EPS - dist, 0.0), 0.0)
            acc_loss = acc_loss + jnp.sum(contrib)
            acc_cnt = acc_cnt + jnp.sum(cand.astype(f32))

    out_ref[0, 0] = acc_loss / jnp.maximum(acc_cnt, 1.0)


def kernel(node_positions, adjacency, edge_index):
    del adjacency
    n = node_positions.shape[1]
    e = edge_index.shape[1]
    pos = node_positions.reshape(n, 2)
    posx = pos[:, 0]
    posy = pos[:, 1]
    src = edge_index[0]
    dst = edge_index[1]
    out = pl.pallas_call(
        _body,
        out_shape=jax.ShapeDtypeStruct((1, 1), jnp.float32),
        out_specs=pl.BlockSpec(memory_space=pltpu.SMEM),
    )(
        posx.reshape(1, n), posy.reshape(1, n),
        posx.reshape(n, 1), posy.reshape(n, 1),
        src.reshape(e, 1), dst.reshape(e, 1),
        src.reshape(1, e), dst.reshape(1, e),
    )
    return out[0, 0]


# speedup vs baseline: 3.4666x; 1.0067x over previous
import jax
import jax.numpy as jnp
from jax.experimental import pallas as pl
from jax.experimental.pallas import tpu as pltpu

EPS = 0.001
PROX = 0.15
RB = 256
CB = 256


def _body(posx_r_ref, posy_r_ref, posx_c_ref, posy_c_ref,
          src_c_ref, dst_c_ref, src_r_ref, dst_r_ref, out_ref):
    f32 = jnp.float32
    posx_r = posx_r_ref[...]
    posy_r = posy_r_ref[...]
    posx_c = posx_c_ref[...]
    posy_c = posy_c_ref[...]
    src_c = src_c_ref[...]
    dst_c = dst_c_ref[...]
    src_r = src_r_ref[...]
    dst_r = dst_r_ref[...]
    n = posx_r.shape[1]
    e = src_c.shape[0]

    gb = 512
    lanes = jax.lax.broadcasted_iota(jnp.int32, (gb, n), 1)
    subl = jax.lax.broadcasted_iota(jnp.int32, (n, gb), 0)

    zero = f32(0.0)

    def gather_col(idx_c):
        xs, ys = [], []
        for g0 in range(0, e, gb):
            m = lanes == idx_c[g0:g0 + gb]
            xs.append(jnp.sum(jnp.where(m, posx_r, zero), axis=1, keepdims=True))
            ys.append(jnp.sum(jnp.where(m, posy_r, zero), axis=1, keepdims=True))
        return jnp.concatenate(xs, axis=0), jnp.concatenate(ys, axis=0)

    def gather_row(idx_r):
        xs, ys = [], []
        for g0 in range(0, e, gb):
            m = subl == idx_r[:, g0:g0 + gb]
            xs.append(jnp.sum(jnp.where(m, posx_c, zero), axis=0, keepdims=True))
            ys.append(jnp.sum(jnp.where(m, posy_c, zero), axis=0, keepdims=True))
        return jnp.concatenate(xs, axis=1), jnp.concatenate(ys, axis=1)

    a1x_c, a1y_c = gather_col(src_c)
    a2x_c, a2y_c = gather_col(dst_c)
    b1x_r, b1y_r = gather_row(src_r)
    b2x_r, b2y_r = gather_row(dst_r)

    d1x_c = a2x_c - a1x_c
    d1y_c = a2y_c - a1y_c
    midx_c = (a1x_c + a2x_c) * 0.5
    midy_c = (a1y_c + a2y_c) * 0.5
    lsq_c = d1x_c * d1x_c + d1y_c * d1y_c
    aa_c = jnp.maximum(lsq_c, 1e-12)
    inv_a_c = 1.0 / aa_c
    hlp_c = jnp.sqrt(jnp.maximum(lsq_c, 1e-24)) * 0.5 + (PROX * 0.5)

    d2x_r = b2x_r - b1x_r
    d2y_r = b2y_r - b1y_r
    midx_r = (b1x_r + b2x_r) * 0.5
    midy_r = (b1y_r + b2y_r) * 0.5
    lsq_r = d2x_r * d2x_r + d2y_r * d2y_r
    ee_r = jnp.maximum(lsq_r, 1e-12)
    inv_e_r = 1.0 / ee_r
    hlp_r = jnp.sqrt(jnp.maximum(lsq_r, 1e-24)) * 0.5 + (PROX * 0.5)

    nb_r = e // RB
    nb_c = e // CB
    acc_loss = f32(0.0)
    acc_cnt = f32(0.0)
    for bi in range(nb_r):
        r0 = bi * RB
        A1x = a1x_c[r0:r0 + RB]
        A1y = a1y_c[r0:r0 + RB]
        D1x = d1x_c[r0:r0 + RB]
        D1y = d1y_c[r0:r0 + RB]
        MIx = midx_c[r0:r0 + RB]
        MIy = midy_c[r0:r0 + RB]
        AA = aa_c[r0:r0 + RB]
        IA = inv_a_c[r0:r0 + RB]
        HI = hlp_c[r0:r0 + RB]
        SI = src_c[r0:r0 + RB]
        DI = dst_c[r0:r0 + RB]
        for bj in range(bi, nb_c):
            c0 = bj * CB
            B1x = b1x_r[:, c0:c0 + CB]
            B1y = b1y_r[:, c0:c0 + CB]
            D2x = d2x_r[:, c0:c0 + CB]
            D2y = d2y_r[:, c0:c0 + CB]
            MJx = midx_r[:, c0:c0 + CB]
            MJy = midy_r[:, c0:c0 + CB]
            EE = ee_r[:, c0:c0 + CB]
            IE = inv_e_r[:, c0:c0 + CB]
            HJ = hlp_r[:, c0:c0 + CB]
            SJ = src_r[:, c0:c0 + CB]
            DJ = dst_r[:, c0:c0 + CB]

            rx = A1x - B1x
            ry = A1y - B1y
            b = D1x * D2x + D1y * D2y
            c = D1x * rx + D1y * ry
            f = D2x * rx + D2y * ry
            denom = jnp.maximum(AA * EE - b * b, 1e-12)
            rden = 1.0 / denom
            s = jnp.clip((b * f - c * EE) * rden, 0.0, 1.0)
            t = jnp.clip((b * s + f) * IE, 0.0, 1.0)
            s = jnp.clip((b * t - c) * IA, 0.0, 1.0)
            dx = rx + s * D1x - t * D2x
            dy = ry + s * D1y - t * D2y
            dist = jnp.sqrt(jnp.maximum(dx * dx + dy * dy, 1e-24))

            mdx = MIx - MJx
            mdy = MIy - MJy
            reach = HI + HJ
            prox = (mdx * mdx + mdy * mdy) < (reach * reach)
            share = ((SI == SJ) | (SI == DJ) | (DI == SJ) | (DI == DJ))
            cand = prox & jnp.logical_not(share)
            if bi == bj:
                ii = jax.lax.broadcasted_iota(jnp.int32, (RB, CB), 0)
                jj = jax.lax.broadcasted_iota(jnp.int32, (RB, CB), 1)
                cand = cand & (jj > ii)
            contrib = jnp.where(cand, jnp.maximum(EPS - dist, 0.0), 0.0)
            acc_loss = acc_loss + jnp.sum(contrib)
            acc_cnt = acc_cnt + jnp.sum(cand.astype(f32))

    out_ref[0, 0] = acc_loss / jnp.maximum(acc_cnt, 1.0)


def kernel(node_positions, adjacency, edge_index):
    del adjacency
    n = node_positions.shape[1]
    e = edge_index.shape[1]
    pos = node_positions.reshape(n, 2)
    posx = pos[:, 0]
    posy = pos[:, 1]
    src = edge_index[0]
    dst = edge_index[1]
    out = pl.pallas_call(
        _body,
        out_shape=jax.ShapeDtypeStruct((1, 1), jnp.float32),
        out_specs=pl.BlockSpec(memory_space=pltpu.SMEM),
    )(
        posx.reshape(1, n), posy.reshape(1, n),
        posx.reshape(n, 1), posy.reshape(n, 1),
        src.reshape(e, 1), dst.reshape(e, 1),
        src.reshape(1, e), dst.reshape(1, e),
    )
    return out[0, 0]
